# Initial kernel scaffold; baseline (speedup 1.0000x reference)
import jax
import jax.numpy as jnp
from jax.experimental import pallas as pl


def kernel(xyz, params):
    raise NotImplementedError("write your pallas kernel here")


# speedup vs baseline: 1.4126x; 1.4126x over previous
import jax
import jax.numpy as jnp
from jax import lax
from jax.experimental import pallas as pl

KNN = 16
UPF = 4
SCALE = 256.0
F32 = jnp.float32
BF16 = jnp.bfloat16


def _mm(a, b):
    return lax.dot_general(a.astype(BF16), b.astype(BF16),
                           (((1,), (0,)), ((), ())),
                           preferred_element_type=F32)


def _gather_mm(oh, fc):
    return lax.dot_general(oh, fc, (((1,), (0,)), ((), ())),
                           precision=lax.Precision.HIGHEST,
                           preferred_element_type=F32)


def _dist2(xr, xaT):
    dot = lax.dot_general(xr.astype(BF16), xaT.astype(BF16),
                          (((1,), (0,)), ((), ())),
                          preferred_element_type=F32)
    sqr = jnp.sum(xr * xr, axis=1, keepdims=True)
    sqa = jnp.sum(xaT * xaT, axis=0, keepdims=True)
    return sqr + sqa - 2.0 * dot


def _extract_min(work, iota, m_cols):
    m = jnp.min(work, axis=1, keepdims=True)
    sel = jnp.min(jnp.where(work == m, iota, m_cols), axis=1, keepdims=True)
    ohb = iota == sel
    work = jnp.where(ohb, jnp.array(jnp.inf, F32), work)
    return ohb.astype(F32), sel, work


def _rnf_from_nb(nb, ctr):
    rel = nb - ctr
    dist = jnp.sqrt(jnp.sum(rel * rel, axis=1, keepdims=True))
    return jnp.concatenate([ctr, nb, rel, dist], axis=1)


def _lfa_h(nbf, rnf, wnb, bnb, wmx, bmx):
    nf = jnp.maximum(_mm(rnf, wnb) + bnb, 0.0)
    hin = jnp.concatenate([nbf, nf], axis=1)
    return jnp.maximum(_mm(hin, wmx) + bmx, 0.0)


def _knn_body(rows_ref, allT_ref, idx_ref):
    xr = rows_ref[0]
    xaT = allT_ref[0]
    m_cols = xaT.shape[1]
    d = _dist2(xr, xaT)
    iota = lax.broadcasted_iota(jnp.int32, d.shape, 1)
    sels = []
    for _ in range(KNN):
        _, sel, d = _extract_min(d, iota, m_cols)
        sels.append(sel)
    idx_ref[0] = jnp.concatenate(sels, axis=1)


def _lfa_body(ctr_ref, idx_ref, fc_ref, wnb_ref, bnb_ref, wmx_ref, bmx_ref,
              out_ref):
    ctr = ctr_ref[0]
    idxb = idx_ref[0]
    fc = fc_ref[0]
    m_cols = fc.shape[0]
    iota = lax.broadcasted_iota(jnp.int32, (ctr.shape[0], m_cols), 1)
    wnb, bnb = wnb_ref[...], bnb_ref[...]
    wmx, bmx = wmx_ref[...], bmx_ref[...]
    acc = None
    for t in range(KNN):
        oh = (iota == idxb[:, t:t + 1]).astype(F32)
        g = _gather_mm(oh, fc)
        rnf = _rnf_from_nb(g[:, :3], ctr)
        h = _lfa_h(g[:, 3:], rnf, wnb, bnb, wmx, bmx)
        acc = h if acc is None else jnp.maximum(acc, h)
    out_ref[0] = acc


def _fin_body(cur_ref, curT_ref, feat_ref, w0nb, b0nb, w0mx, b0mx,
              w1nb, b1nb, w1mx, b1mx, ow_ref, ob_ref, out_ref):
    x = cur_ref[0]
    xT = curT_ref[0]
    f = feat_ref[0]
    m_cols = x.shape[0]
    d = _dist2(x, xT)
    iota = lax.broadcasted_iota(jnp.int32, d.shape, 1)
    fc = jnp.concatenate([x, f], axis=1)
    sels, rnfs = [], []
    acc = None
    for _ in range(KNN):
        oh, sel, d = _extract_min(d, iota, m_cols)
        sels.append(sel)
        g = _gather_mm(oh, fc)
        rnf = _rnf_from_nb(g[:, :3], x)
        rnfs.append(rnf)
        h = _lfa_h(g[:, 3:], rnf, w0nb[...], b0nb[...], w0mx[...], b0mx[...])
        acc = h if acc is None else jnp.maximum(acc, h)
    feat_a = acc
    acc = None
    for t in range(KNN):
        oh = (iota == sels[t]).astype(F32)
        g = _gather_mm(oh, feat_a)
        h = _lfa_h(g, rnfs[t], w1nb[...], b1nb[...], w1mx[...], b1mx[...])
        acc = h if acc is None else jnp.maximum(acc, h)
    y = (_mm(acc, ow_ref[...]) + ob_ref[...]) * SCALE
    yq = y + (jnp.round(y) - y)
    out_ref[0] = yq / SCALE


def _dec_body(ctr_ref, allT_ref, fc_ref, wnb, bnb, wmx, bmx,
              wpred_ref, bpred_ref, f_out_ref, coord_ref):
    ctr = ctr_ref[0]
    xaT = allT_ref[0]
    fc = fc_ref[0]
    m_cols = xaT.shape[1]
    d = _dist2(ctr, xaT)
    iota = lax.broadcasted_iota(jnp.int32, d.shape, 1)
    acc = None
    for _ in range(KNN):
        oh, _, d = _extract_min(d, iota, m_cols)
        g = _gather_mm(oh, fc)
        rnf = _rnf_from_nb(g[:, :3], ctr)
        h = _lfa_h(g[:, 3:], rnf, wnb[...], bnb[...], wmx[...], bmx[...])
        acc = h if acc is None else jnp.maximum(acc, h)
    f_out_ref[0] = acc
    wpred, bpred = wpred_ref[...], bpred_ref[...]
    coords = []
    for c in range(UPF):
        off = _mm(acc[:, 32 * c:32 * (c + 1)], wpred) + bpred
        coords.append(ctr + off)
    coord_ref[0] = jnp.concatenate(coords, axis=1)


def _knn_call(cur, curT, bm):
    B, M, _ = cur.shape
    return pl.pallas_call(
        _knn_body,
        grid=(B, M // bm),
        in_specs=[
            pl.BlockSpec((1, bm, 3), lambda b, i: (b, i, 0)),
            pl.BlockSpec((1, 3, M), lambda b, i: (b, 0, 0)),
        ],
        out_specs=pl.BlockSpec((1, bm, KNN), lambda b, i: (b, i, 0)),
        out_shape=jax.ShapeDtypeStruct((B, M, KNN), jnp.int32),
    )(cur, curT)


def _lfa_call(cur, idx, feat, p, bm):
    B, M, _ = cur.shape
    cin = feat.shape[-1]
    cout = p['wmx'].shape[-1]
    fc = jnp.concatenate([cur, feat], axis=-1)
    return pl.pallas_call(
        _lfa_body,
        grid=(B, M // bm),
        in_specs=[
            pl.BlockSpec((1, bm, 3), lambda b, i: (b, i, 0)),
            pl.BlockSpec((1, bm, KNN), lambda b, i: (b, i, 0)),
            pl.BlockSpec((1, M, 3 + cin), lambda b, i: (b, 0, 0)),
            pl.BlockSpec((10, 16), lambda b, i: (0, 0)),
            pl.BlockSpec((1, 16), lambda b, i: (0, 0)),
            pl.BlockSpec((cin + 16, cout), lambda b, i: (0, 0)),
            pl.BlockSpec((1, cout), lambda b, i: (0, 0)),
        ],
        out_specs=pl.BlockSpec((1, bm, cout), lambda b, i: (b, i, 0)),
        out_shape=jax.ShapeDtypeStruct((B, M, cout), F32),
    )(cur, idx, fc, p['wnb'], p['bnb'].reshape(1, -1),
      p['wmx'], p['bmx'].reshape(1, -1))


def _fin_call(cur, curT, feat, p0, p1, ow, ob):
    B, M, _ = cur.shape
    cin = feat.shape[-1]
    comp = ow.shape[-1]
    args = (cur, curT, feat,
            p0['wnb'], p0['bnb'].reshape(1, -1), p0['wmx'], p0['bmx'].reshape(1, -1),
            p1['wnb'], p1['bnb'].reshape(1, -1), p1['wmx'], p1['bmx'].reshape(1, -1),
            ow, ob.reshape(1, -1))
    return pl.pallas_call(
        _fin_body,
        grid=(B,),
        in_specs=[
            pl.BlockSpec((1, M, 3), lambda b: (b, 0, 0)),
            pl.BlockSpec((1, 3, M), lambda b: (b, 0, 0)),
            pl.BlockSpec((1, M, cin), lambda b: (b, 0, 0)),
        ] + [pl.BlockSpec(a.shape, lambda b: (0, 0)) for a in args[3:]],
        out_specs=pl.BlockSpec((1, M, comp), lambda b: (b, 0, 0)),
        out_shape=jax.ShapeDtypeStruct((B, M, comp), F32),
    )(*args)


def _dec_call(cur, curT, fea, p, bm):
    B, M, _ = cur.shape
    cin = fea.shape[-1]
    cout = p['wmx'].shape[-1]
    fc = jnp.concatenate([cur, fea], axis=-1)
    args = (cur, curT, fc,
            p['wnb'], p['bnb'].reshape(1, -1), p['wmx'], p['bmx'].reshape(1, -1),
            p['wpred'], p['bpred'].reshape(1, -1))
    return pl.pallas_call(
        _dec_body,
        grid=(B, M // bm),
        in_specs=[
            pl.BlockSpec((1, bm, 3), lambda b, i: (b, i, 0)),
            pl.BlockSpec((1, 3, M), lambda b, i: (b, 0, 0)),
            pl.BlockSpec((1, M, 3 + cin), lambda b, i: (b, 0, 0)),
        ] + [pl.BlockSpec(a.shape, lambda b, i: (0, 0)) for a in args[3:]],
        out_specs=[
            pl.BlockSpec((1, bm, cout), lambda b, i: (b, i, 0)),
            pl.BlockSpec((1, bm, 3 * UPF), lambda b, i: (b, i, 0)),
        ],
        out_shape=[
            jax.ShapeDtypeStruct((B, M, cout), F32),
            jax.ShapeDtypeStruct((B, M, 3 * UPF), F32),
        ],
    )(*args)


def kernel(xyz, params):
    B, N, _ = xyz.shape
    cur, feat = xyz, xyz
    curT = jnp.swapaxes(cur, 1, 2)

    for i in range(3):
        bm = min(cur.shape[1], 256)
        curT = jnp.swapaxes(cur, 1, 2)
        idx = _knn_call(cur, curT, bm)
        feat = _lfa_call(cur, idx, feat, params['enc%da' % i], bm)
        feat = _lfa_call(cur, idx, feat, params['enc%db' % i], bm)
        cur, feat = cur[:, ::UPF], feat[:, ::UPF]

    curT = jnp.swapaxes(cur, 1, 2)
    fea = _fin_call(cur, curT, feat, params['fin0'], params['fin1'],
                    params['out_w'], params['out_b'])

    for i in range(3):
        curT = jnp.swapaxes(cur, 1, 2)
        f, coord12 = _dec_call(cur, curT, fea, params['dec%d' % i],
                               min(cur.shape[1], 256))
        M = cur.shape[1]
        fea = f.reshape(B, M * UPF, 32)
        cur = coord12.reshape(B, M * UPF, 3)
    return fea


# speedup vs baseline: 11.0340x; 7.8110x over previous
import functools

import jax
import jax.numpy as jnp
from jax import lax
from jax.experimental import pallas as pl
from jax.experimental.pallas import tpu as pltpu
from jax.experimental.pallas import tpu_sc as plsc

KNN = 16
UPF = 4
SCALE = 256.0
F32 = jnp.float32
BF16 = jnp.bfloat16


def _mm(a, b):
    return lax.dot_general(a.astype(BF16), b.astype(BF16),
                           (((1,), (0,)), ((), ())),
                           preferred_element_type=F32)


def _gather_mm(oh, fc):
    return lax.dot_general(oh, fc, (((1,), (0,)), ((), ())),
                           precision=lax.Precision.HIGHEST,
                           preferred_element_type=F32)


def _dist2(xr, xaT):
    dot = lax.dot_general(xr.astype(BF16), xaT.astype(BF16),
                          (((1,), (0,)), ((), ())),
                          preferred_element_type=F32)
    sqr = jnp.sum(xr * xr, axis=1, keepdims=True)
    sqa = jnp.sum(xaT * xaT, axis=0, keepdims=True)
    return sqr + sqa - 2.0 * dot


def _extract_min(work, iota, m_cols):
    m = jnp.min(work, axis=1, keepdims=True)
    sel = jnp.min(jnp.where(work == m, iota, m_cols), axis=1, keepdims=True)
    ohb = iota == sel
    work = jnp.where(ohb, jnp.array(jnp.inf, F32), work)
    return ohb.astype(F32), sel, work


def _rnf_from_nb(nb, ctr):
    rel = nb - ctr
    dist = jnp.sqrt(jnp.sum(rel * rel, axis=1, keepdims=True))
    return jnp.concatenate([ctr, nb, rel, dist], axis=1)


def _lfa_h(nbf, rnf, wnb, bnb, wmx, bmx):
    nf = jnp.maximum(_mm(rnf, wnb) + bnb, 0.0)
    hin = jnp.concatenate([nbf, nf], axis=1)
    return jnp.maximum(_mm(hin, wmx) + bmx, 0.0)


def _knn_body(rows_ref, allT_ref, idx_ref):
    xr = rows_ref[0]
    xaT = allT_ref[0]
    m_cols = xaT.shape[1]
    d = _dist2(xr, xaT)
    iota = lax.broadcasted_iota(jnp.int32, d.shape, 1)
    sels = []
    for _ in range(KNN):
        _, sel, d = _extract_min(d, iota, m_cols)
        sels.append(sel)
    idx_ref[0] = jnp.concatenate(sels, axis=1) + pl.program_id(0) * m_cols


def _sc_gather(table, idx):
    R, D = table.shape
    btot = idx.shape[0]
    info = plsc.get_sparse_core_info()
    nw = info.num_cores * info.num_subcores
    b_per_w = btot // nw
    chunk = min(b_per_w, 1024)
    nchunks = b_per_w // chunk
    mesh = plsc.VectorSubcoreMesh(core_axis_name="c", subcore_axis_name="s")

    @functools.partial(
        pl.kernel, mesh=mesh,
        out_type=jax.ShapeDtypeStruct((btot, D), F32),
        compiler_params=pltpu.CompilerParams(use_tc_tiling_on_sc=False),
        scratch_types=[
            pltpu.VMEM((chunk,), jnp.int32),
            pltpu.VMEM((chunk, D), F32),
            pltpu.SemaphoreType.DMA,
        ],
    )
    def k(table_hbm, idx_hbm, out_hbm, idx_v, rows_v, sem):
        wid = lax.axis_index("s") * info.num_cores + lax.axis_index("c")
        base = wid * b_per_w

        def body(c, carry):
            off = base + c * chunk
            pltpu.sync_copy(idx_hbm.at[pl.ds(off, chunk)], idx_v)
            pltpu.async_copy(table_hbm.at[idx_v], rows_v, sem).wait()
            pltpu.sync_copy(rows_v, out_hbm.at[pl.ds(off, chunk)])
            return carry

        if nchunks == 1:
            body(0, 0)
        else:
            lax.fori_loop(0, nchunks, body, 0)

    return k(table, idx)


def _lfa2_body(ctr_ref, g_ref, wnb_ref, bnb_ref, wmx_ref, bmx_ref, out_ref):
    ctr = ctr_ref[...]
    g = g_ref[...]
    bm = ctr.shape[0]
    ctrk = jnp.broadcast_to(ctr[:, None, :], (bm, KNN, 3)).reshape(bm * KNN, 3)
    rnf = _rnf_from_nb(g[:, :3], ctrk)
    h = _lfa_h(g[:, 3:3 + (wmx_ref.shape[0] - 16)], rnf,
               wnb_ref[...], bnb_ref[...], wmx_ref[...], bmx_ref[...])
    out_ref[...] = jnp.max(h.reshape(bm, KNN, h.shape[1]), axis=1)


def _dec2_body(ctr_ref, g_ref, wnb_ref, bnb_ref, wmx_ref, bmx_ref,
               wpred_ref, bpred_ref, f_out_ref, coord_ref):
    ctr = ctr_ref[...]
    g = g_ref[...]
    bm = ctr.shape[0]
    ctrk = jnp.broadcast_to(ctr[:, None, :], (bm, KNN, 3)).reshape(bm * KNN, 3)
    rnf = _rnf_from_nb(g[:, :3], ctrk)
    h = _lfa_h(g[:, 3:3 + (wmx_ref.shape[0] - 16)], rnf,
               wnb_ref[...], bnb_ref[...], wmx_ref[...], bmx_ref[...])
    acc = jnp.max(h.reshape(bm, KNN, h.shape[1]), axis=1)
    f_out_ref[...] = acc
    wpred, bpred = wpred_ref[...], bpred_ref[...]
    coords = []
    for c in range(UPF):
        off = _mm(acc[:, 32 * c:32 * (c + 1)], wpred) + bpred
        coords.append(ctr + off)
    coord_ref[...] = jnp.concatenate(coords, axis=1)


def _fin_body(cur_ref, curT_ref, feat_ref, w0nb, b0nb, w0mx, b0mx,
              w1nb, b1nb, w1mx, b1mx, ow_ref, ob_ref, out_ref):
    x = cur_ref[0]
    xT = curT_ref[0]
    f = feat_ref[0]
    m_cols = x.shape[0]
    d = _dist2(x, xT)
    iota = lax.broadcasted_iota(jnp.int32, d.shape, 1)
    fc = jnp.concatenate([x, f], axis=1)
    sels, rnfs = [], []
    acc = None
    for _ in range(KNN):
        oh, sel, d = _extract_min(d, iota, m_cols)
        sels.append(sel)
        g = _gather_mm(oh, fc)
        rnf = _rnf_from_nb(g[:, :3], x)
        rnfs.append(rnf)
        h = _lfa_h(g[:, 3:], rnf, w0nb[...], b0nb[...], w0mx[...], b0mx[...])
        acc = h if acc is None else jnp.maximum(acc, h)
    feat_a = acc
    acc = None
    for t in range(KNN):
        oh = (iota == sels[t]).astype(F32)
        g = _gather_mm(oh, feat_a)
        h = _lfa_h(g, rnfs[t], w1nb[...], b1nb[...], w1mx[...], b1mx[...])
        acc = h if acc is None else jnp.maximum(acc, h)
    y = (_mm(acc, ow_ref[...]) + ob_ref[...]) * SCALE
    yq = y + (jnp.round(y) - y)
    out_ref[0] = yq / SCALE


def _knn_call(cur, curT, bm):
    B, M, _ = cur.shape
    return pl.pallas_call(
        _knn_body,
        grid=(B, M // bm),
        in_specs=[
            pl.BlockSpec((1, bm, 3), lambda b, i: (b, i, 0)),
            pl.BlockSpec((1, 3, M), lambda b, i: (b, 0, 0)),
        ],
        out_specs=pl.BlockSpec((1, bm, KNN), lambda b, i: (b, i, 0)),
        out_shape=jax.ShapeDtypeStruct((B, M, KNN), jnp.int32),
    )(cur, curT)


def _lfa2_call(cur_flat, g, p, bm):
    btot = cur_flat.shape[0]
    cout = p['wmx'].shape[-1]
    dcols = g.shape[1]
    return pl.pallas_call(
        _lfa2_body,
        grid=(btot // bm,),
        in_specs=[
            pl.BlockSpec((bm, 3), lambda j: (j, 0)),
            pl.BlockSpec((bm * KNN, dcols), lambda j: (j, 0)),
            pl.BlockSpec((10, 16), lambda j: (0, 0)),
            pl.BlockSpec((1, 16), lambda j: (0, 0)),
            pl.BlockSpec(p['wmx'].shape, lambda j: (0, 0)),
            pl.BlockSpec((1, cout), lambda j: (0, 0)),
        ],
        out_specs=pl.BlockSpec((bm, cout), lambda j: (j, 0)),
        out_shape=jax.ShapeDtypeStruct((btot, cout), F32),
    )(cur_flat, g, p['wnb'], p['bnb'].reshape(1, -1),
      p['wmx'], p['bmx'].reshape(1, -1))


def _dec2_call(cur_flat, g, p, bm):
    btot = cur_flat.shape[0]
    cout = p['wmx'].shape[-1]
    dcols = g.shape[1]
    return pl.pallas_call(
        _dec2_body,
        grid=(btot // bm,),
        in_specs=[
            pl.BlockSpec((bm, 3), lambda j: (j, 0)),
            pl.BlockSpec((bm * KNN, dcols), lambda j: (j, 0)),
            pl.BlockSpec((10, 16), lambda j: (0, 0)),
            pl.BlockSpec((1, 16), lambda j: (0, 0)),
            pl.BlockSpec(p['wmx'].shape, lambda j: (0, 0)),
            pl.BlockSpec((1, cout), lambda j: (0, 0)),
            pl.BlockSpec(p['wpred'].shape, lambda j: (0, 0)),
            pl.BlockSpec((1, 3), lambda j: (0, 0)),
        ],
        out_specs=[
            pl.BlockSpec((bm, cout), lambda j: (j, 0)),
            pl.BlockSpec((bm, 3 * UPF), lambda j: (j, 0)),
        ],
        out_shape=[
            jax.ShapeDtypeStruct((btot, cout), F32),
            jax.ShapeDtypeStruct((btot, 3 * UPF), F32),
        ],
    )(cur_flat, g, p['wnb'], p['bnb'].reshape(1, -1),
      p['wmx'], p['bmx'].reshape(1, -1),
      p['wpred'], p['bpred'].reshape(1, -1))


def _pad16(x):
    pad = (-x.shape[-1]) % 16
    if pad:
        x = jnp.pad(x, ((0, 0), (0, pad)))
    return x


def _fin_call(cur, curT, feat, p0, p1, ow, ob):
    B, M, _ = cur.shape
    cin = feat.shape[-1]
    comp = ow.shape[-1]
    args = (cur, curT, feat,
            p0['wnb'], p0['bnb'].reshape(1, -1), p0['wmx'], p0['bmx'].reshape(1, -1),
            p1['wnb'], p1['bnb'].reshape(1, -1), p1['wmx'], p1['bmx'].reshape(1, -1),
            ow, ob.reshape(1, -1))
    return pl.pallas_call(
        _fin_body,
        grid=(B,),
        in_specs=[
            pl.BlockSpec((1, M, 3), lambda b: (b, 0, 0)),
            pl.BlockSpec((1, 3, M), lambda b: (b, 0, 0)),
            pl.BlockSpec((1, M, cin), lambda b: (b, 0, 0)),
        ] + [pl.BlockSpec(a.shape, lambda b: (0, 0)) for a in args[3:]],
        out_specs=pl.BlockSpec((1, M, comp), lambda b: (b, 0, 0)),
        out_shape=jax.ShapeDtypeStruct((B, M, comp), F32),
    )(*args)


def kernel(xyz, params):
    B, N, _ = xyz.shape
    cur, feat = xyz, xyz

    for i in range(3):
        M = cur.shape[1]
        curT = jnp.swapaxes(cur, 1, 2)
        idx = _knn_call(cur, curT, min(M, 256))
        idxf = idx.reshape(-1)
        cur_flat = cur.reshape(B * M, 3)
        bm = min(B * M, 256)
        for half in ('a', 'b'):
            table = _pad16(jnp.concatenate(
                [cur_flat, feat.reshape(B * M, -1)], axis=-1))
            g = _sc_gather(table, idxf)
            feat = _lfa2_call(cur_flat, g, params['enc%d%s' % (i, half)], bm)
        feat = feat.reshape(B, M, -1)
        cur, feat = cur[:, ::UPF], feat[:, ::UPF]

    curT = jnp.swapaxes(cur, 1, 2)
    fea = _fin_call(cur, curT, feat, params['fin0'], params['fin1'],
                    params['out_w'], params['out_b'])

    for i in range(3):
        M = cur.shape[1]
        curT = jnp.swapaxes(cur, 1, 2)
        idx = _knn_call(cur, curT, min(M, 256))
        idxf = idx.reshape(-1)
        cur_flat = cur.reshape(B * M, 3)
        table = _pad16(jnp.concatenate(
            [cur_flat, fea.reshape(B * M, -1)], axis=-1))
        g = _sc_gather(table, idxf)
        f, coord12 = _dec2_call(cur_flat, g, params['dec%d' % i],
                                min(B * M, 256))
        fea = f.reshape(B, M * UPF, 32)
        cur = coord12.reshape(B, M * UPF, 3)
    return fea
